# Initial kernel scaffold; baseline (speedup 1.0000x reference)
import jax
import jax.numpy as jnp
from jax.experimental import pallas as pl


def kernel(x):
    raise NotImplementedError("write your pallas kernel here")


# speedup vs baseline: 1.8789x; 1.8789x over previous
import functools

import jax
import jax.numpy as jnp
from jax import lax
from jax.experimental import pallas as pl
from jax.experimental.pallas import tpu as pltpu
from jax.experimental.pallas import tpu_sc as plsc

_L = 16
_NC = 2
_NS = 16
_NW = _NC * _NS

_N_ELEMS = 1048576 * 2
_PER_W = _N_ELEMS // _NW

_mesh = plsc.VectorSubcoreMesh(core_axis_name="c", subcore_axis_name="s")


@functools.partial(
    pl.kernel,
    mesh=_mesh,
    out_type=jax.ShapeDtypeStruct((_N_ELEMS,), jnp.float32),
    scratch_types=[pltpu.VMEM((_PER_W,), jnp.float32)],
    compiler_params=pltpu.CompilerParams(needs_layout_passes=False),
)
def _hardmax_sc(x_hbm, out_hbm, buf):
    wid = lax.axis_index("s") * _NC + lax.axis_index("c")
    base = wid * _PER_W
    pltpu.sync_copy(x_hbm.at[pl.ds(base, _PER_W)], buf)

    lane = lax.iota(jnp.int32, _L)

    def step(i, carry):
        off = i * (2 * _L)
        idx_a = off + 2 * lane
        idx_b = idx_a + 1
        a = plsc.load_gather(buf, [idx_a])
        b = plsc.load_gather(buf, [idx_b])
        ha = jnp.where(a >= b, 1.0, 0.0).astype(jnp.float32)
        hb = 1.0 - ha
        plsc.store_scatter(buf, [idx_a], (ha - a) + a)
        plsc.store_scatter(buf, [idx_b], (hb - b) + b)
        return carry

    lax.fori_loop(0, _PER_W // (2 * _L), step, 0)

    pltpu.sync_copy(buf, out_hbm.at[pl.ds(base, _PER_W)])


def kernel(x):
    y_flat = _hardmax_sc(x.reshape(-1))
    return y_flat.reshape(x.shape)


# speedup vs baseline: 142.0256x; 75.5907x over previous
import functools

import jax
import jax.numpy as jnp
from jax import lax
from jax.experimental import pallas as pl
from jax.experimental.pallas import tpu as pltpu
from jax.experimental.pallas import tpu_sc as plsc

_L = 16
_NC = 2
_NS = 16
_NW = _NC * _NS

_N_ROWS = 1048576
_GROUP = 2 * 128
_N_ELEMS = _N_ROWS * 2
_PER_W = _N_ELEMS // _NW
_GROUPS_W = _PER_W // _GROUP

_mesh = plsc.VectorSubcoreMesh(core_axis_name="c", subcore_axis_name="s")


@functools.partial(
    pl.kernel,
    mesh=_mesh,
    out_type=jax.ShapeDtypeStruct((_N_ELEMS,), jnp.float32),
    scratch_types=[pltpu.VMEM((_PER_W,), jnp.float32)],
    compiler_params=pltpu.CompilerParams(needs_layout_passes=False),
)
def _hardmax_sc(x_hbm, out_hbm, buf):
    wid = lax.axis_index("s") * _NC + lax.axis_index("c")
    base = wid * _PER_W
    pltpu.sync_copy(x_hbm.at[pl.ds(base, _PER_W)], buf)

    def group_step(g, carry):
        goff = g * _GROUP
        for u in range(128 // _L):
            a_off = goff + u * _L
            b_off = a_off + 128
            a = buf[pl.ds(a_off, _L)]
            b = buf[pl.ds(b_off, _L)]
            ha = jnp.where(a >= b, 1.0, 0.0).astype(jnp.float32)
            hb = 1.0 - ha
            buf[pl.ds(a_off, _L)] = (ha - a) + a
            buf[pl.ds(b_off, _L)] = (hb - b) + b
        return carry

    lax.fori_loop(0, _GROUPS_W, group_step, 0)

    pltpu.sync_copy(buf, out_hbm.at[pl.ds(base, _PER_W)])


def kernel(x):
    n = x.shape[0]
    xg = x.reshape(n // 128, 128, 2).transpose(0, 2, 1)
    y_flat = _hardmax_sc(xg.reshape(-1))
    yg = y_flat.reshape(n // 128, 2, 128).transpose(0, 2, 1)
    return yg.reshape(n, 2)


# speedup vs baseline: 144.1388x; 1.0149x over previous
import functools

import jax
import jax.numpy as jnp
from jax import lax
from jax.experimental import pallas as pl
from jax.experimental.pallas import tpu as pltpu
from jax.experimental.pallas import tpu_sc as plsc

_L = 16
_NC = 2
_NS = 16
_NW = _NC * _NS

_N_ROWS = 1048576
_GROUP = 2 * 128
_N_ELEMS = _N_ROWS * 2
_PER_W = _N_ELEMS // _NW
_GROUPS_W = _PER_W // _GROUP

_mesh = plsc.VectorSubcoreMesh(core_axis_name="c", subcore_axis_name="s")


_NSLOTS = 4
_CHUNK = 8192
_NCHUNKS = _PER_W // _CHUNK
_GROUPS_C = _CHUNK // _GROUP


@functools.partial(
    pl.kernel,
    mesh=_mesh,
    out_type=jax.ShapeDtypeStruct((_N_ELEMS,), jnp.float32),
    scratch_types=[pltpu.VMEM((_CHUNK,), jnp.float32)] * _NSLOTS
    + [pltpu.SemaphoreType.DMA] * (2 * _NSLOTS),
    compiler_params=pltpu.CompilerParams(needs_layout_passes=False),
)
def _hardmax_sc(x_hbm, out_hbm, *scratch):
    buf = scratch[:_NSLOTS]
    sin, sout = scratch[_NSLOTS:2 * _NSLOTS], scratch[2 * _NSLOTS:]
    wid = lax.axis_index("s") * _NC + lax.axis_index("c")
    base = wid * _PER_W

    def in_dma(c):
        s = c % _NSLOTS
        return pltpu.make_async_copy(
            x_hbm.at[pl.ds(base + c * _CHUNK, _CHUNK)], buf[s], sin[s])

    def out_dma(c):
        s = c % _NSLOTS
        return pltpu.make_async_copy(
            buf[s], out_hbm.at[pl.ds(base + c * _CHUNK, _CHUNK)], sout[s])

    def compute(s):
        bref = buf[s]

        def group_step(g, carry):
            goff = g * _GROUP
            for u in range(128 // _L):
                a_off = goff + u * _L
                b_off = a_off + 128
                a = bref[pl.ds(a_off, _L)]
                b = bref[pl.ds(b_off, _L)]
                ha = jnp.where(a >= b, 1.0, 0.0).astype(jnp.float32)
                hb = 1.0 - ha
                bref[pl.ds(a_off, _L)] = (ha - a) + a
                bref[pl.ds(b_off, _L)] = (hb - b) + b
            return carry

        lax.fori_loop(0, _GROUPS_C, group_step, 0)

    in_dma(0).start()
    in_dma(1).start()
    for c in range(_NCHUNKS):
        if c + 2 < _NCHUNKS:
            if c - 2 >= 0:
                out_dma(c - 2).wait()
            in_dma(c + 2).start()
        in_dma(c).wait()
        compute(c % _NSLOTS)
        out_dma(c).start()
    out_dma(_NCHUNKS - 2).wait()
    out_dma(_NCHUNKS - 1).wait()


def kernel(x):
    n = x.shape[0]
    xg = x.reshape(n // 128, 128, 2).transpose(0, 2, 1)
    y_flat = _hardmax_sc(xg.reshape(-1))
    yg = y_flat.reshape(n // 128, 2, 128).transpose(0, 2, 1)
    return yg.reshape(n, 2)
